# baseline (speedup 1.0000x reference)
import functools

import jax
import jax.numpy as jnp
from jax import lax
from jax.experimental import pallas as pl
from jax.experimental.pallas import tpu as pltpu
from jax.experimental.pallas import tpu_sc as plsc

_KS = 25
_NBITS = 12
_CH = 128


def _decomp_kernel(x_ref, cs_ref, scan_ref):
    x = x_ref[0]
    C, T = x.shape
    npad_l = _KS // 2
    n_in = T + 2 * npad_l
    nch = (n_in + _CH - 1) // _CH
    npad_r = nch * _CH - T - npad_l

    zl = jnp.zeros((C, npad_l), jnp.float32)
    zr = jnp.zeros((C, npad_r), jnp.float32)
    xp = jnp.concatenate([zl, x, zr], axis=1)

    cols = [xp[:, c * _CH:(c + 1) * _CH].T for c in range(nch)]
    scan_ref[...] = jnp.concatenate(cols, axis=1)

    def body(r, carry):
        nxt = carry + scan_ref[pl.ds(r, 1), :]
        scan_ref[pl.ds(r, 1), :] = nxt
        return nxt

    totals = jax.lax.fori_loop(1, _CH, body, scan_ref[pl.ds(0, 1), :])

    offs = [jnp.zeros((1, C), jnp.float32)]
    for c in range(1, nch):
        offs.append(offs[-1] + totals[:, (c - 1) * C:c * C])

    cs_cols = []
    for c in range(nch):
        blk = scan_ref[:, c * C:(c + 1) * C] + offs[c]
        cs_cols.append(blk.T)
    cs = jnp.concatenate(cs_cols, axis=1)

    cs_ref[0] = cs


def _first_argmax(a, fill):
    m = jnp.max(a, axis=-1, keepdims=True)
    idx = jax.lax.broadcasted_iota(jnp.int32, a.shape, 1)
    only = jnp.where(a == m, idx, fill)
    return jnp.min(only, axis=-1, keepdims=True)


def _select_kernel(ac_ref, lag_ref):
    ac = ac_ref[...]
    C, T = ac.shape

    lane = jax.lax.broadcasted_iota(jnp.int32, ac.shape, 1)
    neginf = jnp.float32(-jnp.inf)
    cand = jnp.where(lane == 0, neginf, ac)
    m1 = _first_argmax(cand, T)
    cand = jnp.where(lane == m1, neginf, cand)
    m2 = _first_argmax(cand, T)

    lag_ref[...] = jnp.concatenate(
        [jnp.broadcast_to(m1, (C, 16)), jnp.broadcast_to(m2, (C, 16)),
         jnp.zeros((C, 96), jnp.int32)], axis=1)


def _sc_shift_body(nc, ns, rows_per, s_hbm, lag_hbm, out_hbm,
                   row_v, out_v, lag_v):
    T = 4096
    wid = lax.axis_index("s") * nc + lax.axis_index("c")
    zero16 = jnp.zeros((16,), jnp.float32)

    @pl.loop(0, rows_per)
    def _row(i):
        r = wid * rows_per + i
        pltpu.sync_copy(s_hbm.at[r], row_v)
        pltpu.sync_copy(lag_hbm.at[r], lag_v)
        l1 = lag_v[pl.ds(0, 16)]
        l2 = lag_v[pl.ds(16, 16)]
        iota = lax.iota(jnp.int32, 16)

        @pl.loop(0, T // 16, unroll=4)
        def _chunk(c):
            base = c * 16
            bidx = base + iota
            i1 = bidx - l1
            i2 = bidx - l2
            g1 = plsc.load_gather(row_v, [jnp.maximum(i1, 0)])
            g2 = plsc.load_gather(row_v, [jnp.maximum(i2, 0)])
            a = (row_v[pl.ds(base, 16)] + jnp.where(i1 >= 0, g1, zero16)
                 + jnp.where(i2 >= 0, g2, zero16)) * (1.0 / 3.0)
            out_v[pl.ds(base, 16)] = a

        pltpu.sync_copy(out_v, out_hbm.at[r])


def _mix_kernel(agg_ref, trend_ref, w_ref, b_ref, out_ref):
    out_ref[0] = jax.lax.dot_general(
        w_ref[...], agg_ref[0] + trend_ref[0], (((1,), (0,)), ((), ())),
        preferred_element_type=jnp.float32,
        precision=jax.lax.Precision.HIGHEST) + b_ref[...]


def kernel(x, W, b):
    B, C, T = x.shape
    npad_l = _KS // 2
    n_in = T + 2 * npad_l
    nch = (n_in + _CH - 1) // _CH

    bs = pl.BlockSpec((1, C, T), lambda i: (i, 0, 0))

    cs_full = pl.pallas_call(
        _decomp_kernel,
        grid=(B,),
        in_specs=[bs],
        out_specs=pl.BlockSpec((1, C, nch * _CH), lambda i: (i, 0, 0)),
        out_shape=jax.ShapeDtypeStruct((B, C, nch * _CH), jnp.float32),
        scratch_shapes=[pltpu.VMEM((_CH, nch * C), jnp.float32)],
    )(x)

    cs = cs_full[..., :n_in]
    zero = jnp.zeros(cs.shape[:-1] + (1,), cs.dtype)
    cs = jnp.concatenate([zero, cs], axis=-1)
    trend = (cs[..., _KS:] - cs[..., :-_KS]) / _KS
    s_e = x - trend
    fx = jnp.fft.rfft(s_e, axis=-1)
    ac = jnp.fft.irfft(fx * jnp.conj(fx), n=T, axis=-1)

    R = B * C
    ac2 = ac.reshape(R, T)
    s2 = s_e.reshape(R, T)
    lags = pl.pallas_call(
        _select_kernel,
        grid=(B,),
        in_specs=[pl.BlockSpec((C, T), lambda i: (i, 0))],
        out_specs=pl.BlockSpec((C, 128), lambda i: (i, 0)),
        out_shape=jax.ShapeDtypeStruct((R, 128), jnp.int32),
    )(ac2)

    info = plsc.get_sparse_core_info()
    nc, ns = info.num_cores, info.num_subcores
    rows_per = R // (nc * ns)
    mesh = plsc.VectorSubcoreMesh(core_axis_name="c", subcore_axis_name="s")
    sc_shift = pl.kernel(
        functools.partial(_sc_shift_body, nc, ns, rows_per),
        out_type=jax.ShapeDtypeStruct((R, T), jnp.float32),
        mesh=mesh,
        compiler_params=pltpu.CompilerParams(needs_layout_passes=False),
        scratch_types=[
            pltpu.VMEM((T,), jnp.float32),
            pltpu.VMEM((T,), jnp.float32),
            pltpu.VMEM((128,), jnp.int32),
        ],
    )
    agg2 = sc_shift(s2, lags)
    agg = agg2.reshape(B, C, T)

    out = pl.pallas_call(
        _mix_kernel,
        grid=(B,),
        in_specs=[
            bs, bs,
            pl.BlockSpec((C, C), lambda i: (0, 0)),
            pl.BlockSpec((C, 1), lambda i: (0, 0)),
        ],
        out_specs=bs,
        out_shape=jax.ShapeDtypeStruct((B, C, T), jnp.float32),
    )(agg, trend, W, b.reshape(C, 1))
    return out, trend, agg


# speedup vs baseline: 1.0008x; 1.0008x over previous
import functools

import jax
import jax.numpy as jnp
from jax import lax
from jax.experimental import pallas as pl
from jax.experimental.pallas import tpu as pltpu
from jax.experimental.pallas import tpu_sc as plsc

_KS = 25
_CH = 128


def _decomp_kernel(x_ref, cs_ref, scan_ref):
    x = x_ref[0]
    C, T = x.shape
    npad_l = _KS // 2
    n_in = T + 2 * npad_l
    nch = (n_in + _CH - 1) // _CH
    npad_r = nch * _CH - T - npad_l

    zl = jnp.zeros((C, npad_l), jnp.float32)
    zr = jnp.zeros((C, npad_r), jnp.float32)
    xp = jnp.concatenate([zl, x, zr], axis=1)

    cols = [xp[:, c * _CH:(c + 1) * _CH].T for c in range(nch)]
    scan_ref[...] = jnp.concatenate(cols, axis=1)

    def body(r, carry):
        nxt = carry + scan_ref[pl.ds(r, 1), :]
        scan_ref[pl.ds(r, 1), :] = nxt
        return nxt

    totals = jax.lax.fori_loop(1, _CH, body, scan_ref[pl.ds(0, 1), :])

    offs = [jnp.zeros((1, C), jnp.float32)]
    for c in range(1, nch):
        offs.append(offs[-1] + totals[:, (c - 1) * C:c * C])

    cs_cols = []
    for c in range(nch):
        blk = scan_ref[:, c * C:(c + 1) * C] + offs[c]
        cs_cols.append(blk.T)
    cs = jnp.concatenate(cs_cols, axis=1)

    cs_ref[0] = cs


def _first_argmax(a, fill):
    m = jnp.max(a, axis=-1, keepdims=True)
    idx = jax.lax.broadcasted_iota(jnp.int32, a.shape, 1)
    only = jnp.where(a == m, idx, fill)
    return jnp.min(only, axis=-1, keepdims=True)


def _select_kernel(ac_ref, lag_ref):
    ac = ac_ref[...]
    C, T = ac.shape

    lane = jax.lax.broadcasted_iota(jnp.int32, ac.shape, 1)
    neginf = jnp.float32(-jnp.inf)
    cand = jnp.where(lane == 0, neginf, ac)
    m1 = _first_argmax(cand, T)
    cand = jnp.where(lane == m1, neginf, cand)
    m2 = _first_argmax(cand, T)

    lag_ref[...] = jnp.concatenate(
        [jnp.broadcast_to(m1, (C, 16)), jnp.broadcast_to(m2, (C, 16)),
         jnp.zeros((C, 96), jnp.int32)], axis=1)


def _sc_shift_body(nc, ns, rows_per, s_hbm, lag_hbm, out_hbm,
                   row_v, out_v, lag_v):
    T = 4096
    wid = lax.axis_index("s") * nc + lax.axis_index("c")
    zero16 = jnp.zeros((16,), jnp.float32)

    @pl.loop(0, rows_per)
    def _row(i):
        r = wid * rows_per + i
        pltpu.sync_copy(s_hbm.at[r], row_v)
        pltpu.sync_copy(lag_hbm.at[r], lag_v)
        l1 = lag_v[pl.ds(0, 16)]
        l2 = lag_v[pl.ds(16, 16)]
        iota = lax.iota(jnp.int32, 16)

        @pl.loop(0, T // 16, unroll=4)
        def _chunk(c):
            base = c * 16
            bidx = base + iota
            i1 = bidx - l1
            i2 = bidx - l2
            g1 = plsc.load_gather(row_v, [jnp.maximum(i1, 0)])
            g2 = plsc.load_gather(row_v, [jnp.maximum(i2, 0)])
            a = (row_v[pl.ds(base, 16)] + jnp.where(i1 >= 0, g1, zero16)
                 + jnp.where(i2 >= 0, g2, zero16)) * (1.0 / 3.0)
            out_v[pl.ds(base, 16)] = a

        pltpu.sync_copy(out_v, out_hbm.at[r])


def _mix_kernel(agg_ref, trend_ref, w_ref, b_ref, out_ref):
    out_ref[0] = jax.lax.dot_general(
        w_ref[...], agg_ref[0] + trend_ref[0], (((1,), (0,)), ((), ())),
        preferred_element_type=jnp.float32,
        precision=jax.lax.Precision.HIGHEST) + b_ref[...]


def kernel(x, W, b):
    B, C, T = x.shape
    npad_l = _KS // 2
    n_in = T + 2 * npad_l
    nch = (n_in + _CH - 1) // _CH

    bs = pl.BlockSpec((1, C, T), lambda i: (i, 0, 0))

    cs_full = pl.pallas_call(
        _decomp_kernel,
        grid=(B,),
        in_specs=[bs],
        out_specs=pl.BlockSpec((1, C, nch * _CH), lambda i: (i, 0, 0)),
        out_shape=jax.ShapeDtypeStruct((B, C, nch * _CH), jnp.float32),
        scratch_shapes=[pltpu.VMEM((_CH, nch * C), jnp.float32)],
    )(x)

    cs = cs_full[..., :n_in]
    zero = jnp.zeros(cs.shape[:-1] + (1,), cs.dtype)
    cs = jnp.concatenate([zero, cs], axis=-1)
    trend = (cs[..., _KS:] - cs[..., :-_KS]) / _KS
    s_e = x - trend
    fx = jnp.fft.rfft(s_e, axis=-1)
    ac = jnp.fft.irfft(fx * jnp.conj(fx), n=T, axis=-1)

    R = B * C
    ac2 = ac.reshape(R, T)
    s2 = s_e.reshape(R, T)
    lags = pl.pallas_call(
        _select_kernel,
        grid=(B,),
        in_specs=[pl.BlockSpec((C, T), lambda i: (i, 0))],
        out_specs=pl.BlockSpec((C, 128), lambda i: (i, 0)),
        out_shape=jax.ShapeDtypeStruct((R, 128), jnp.int32),
    )(ac2)

    info = plsc.get_sparse_core_info()
    nc, ns = info.num_cores, info.num_subcores
    rows_per = R // (nc * ns)
    mesh = plsc.VectorSubcoreMesh(core_axis_name="c", subcore_axis_name="s")
    sc_shift = pl.kernel(
        functools.partial(_sc_shift_body, nc, ns, rows_per),
        out_type=jax.ShapeDtypeStruct((R, T), jnp.float32),
        mesh=mesh,
        compiler_params=pltpu.CompilerParams(needs_layout_passes=False),
        scratch_types=[
            pltpu.VMEM((T,), jnp.float32),
            pltpu.VMEM((T,), jnp.float32),
            pltpu.VMEM((128,), jnp.int32),
        ],
    )
    agg2 = sc_shift(s2, lags)
    agg = agg2.reshape(B, C, T)

    out = pl.pallas_call(
        _mix_kernel,
        grid=(B,),
        in_specs=[
            bs, bs,
            pl.BlockSpec((C, C), lambda i: (0, 0)),
            pl.BlockSpec((C, 1), lambda i: (0, 0)),
        ],
        out_specs=bs,
        out_shape=jax.ShapeDtypeStruct((B, C, T), jnp.float32),
    )(agg, trend, W, b.reshape(C, 1))
    return out, trend, agg
